# Initial kernel scaffold; baseline (speedup 1.0000x reference)
import jax
import jax.numpy as jnp
from jax.experimental import pallas as pl


def kernel(input, weight):
    raise NotImplementedError("write your pallas kernel here")


# speedup vs baseline: 4.0873x; 4.0873x over previous
import functools

import jax
import jax.numpy as jnp
from jax import lax
from jax.experimental import pallas as pl
from jax.experimental.pallas import tpu as pltpu
from jax.experimental.pallas import tpu_sc as plsc

BATCH = 4096
HIST = 50
N_D = 64
B = BATCH * HIST
NC, NS = 2, 16
NW = NC * NS
CH = 128
NCHUNK = B // (NW * CH)


@functools.partial(
    pl.kernel,
    out_type=jax.ShapeDtypeStruct((B, N_D), jnp.float32),
    mesh=plsc.VectorSubcoreMesh(core_axis_name="c", subcore_axis_name="s"),
    scratch_types=[
        pltpu.VMEM((NCHUNK, CH), jnp.int32),
        pltpu.VMEM((CH, N_D), jnp.float32),
        pltpu.SemaphoreType.DMA,
    ],
    compiler_params=pltpu.CompilerParams(use_tc_tiling_on_sc=False),
)
def _emb_lookup(idx_hbm, table_hbm, out_hbm, idx_v, rows_v, sem):
    wid = lax.axis_index("s") * NC + lax.axis_index("c")
    base = wid * (NCHUNK * CH)
    pltpu.sync_copy(idx_hbm.at[wid], idx_v)

    def body(c, _):
        pltpu.async_copy(table_hbm.at[idx_v.at[c]], rows_v, sem).wait()
        pltpu.sync_copy(rows_v, out_hbm.at[pl.ds(base + c * CH, CH)])
        return ()

    lax.fori_loop(0, NCHUNK, body, (), unroll=False)


def kernel(input, weight):
    idx = input.astype(jnp.int32).reshape(NW, NCHUNK, CH)
    out = _emb_lookup(idx, weight)
    return out.reshape(BATCH, HIST, N_D)


# speedup vs baseline: 4.6553x; 1.1390x over previous
import functools

import jax
import jax.numpy as jnp
from jax import lax
from jax.experimental import pallas as pl
from jax.experimental.pallas import tpu as pltpu
from jax.experimental.pallas import tpu_sc as plsc

BATCH = 4096
HIST = 50
N_D = 64
B = BATCH * HIST
NC, NS = 2, 16
NW = NC * NS
CH = 128
NCHUNK = B // (NW * CH)
G = 5
NGRP = NCHUNK // G
ROWS_G = G * CH


@functools.partial(
    pl.kernel,
    out_type=jax.ShapeDtypeStruct((B, N_D), jnp.float32),
    mesh=plsc.VectorSubcoreMesh(core_axis_name="c", subcore_axis_name="s"),
    scratch_types=[
        pltpu.VMEM((NCHUNK, CH), jnp.int32),
        pltpu.VMEM((2, ROWS_G, N_D), jnp.float32),
        pltpu.SemaphoreType.DMA,
        pltpu.SemaphoreType.DMA,
        pltpu.SemaphoreType.DMA,
        pltpu.SemaphoreType.DMA,
    ],
    compiler_params=pltpu.CompilerParams(use_tc_tiling_on_sc=False),
)
def _emb_lookup(idx_hbm, table_hbm, out_hbm, idx_v, big, g0, g1, w0, w1):
    wid = lax.axis_index("s") * NC + lax.axis_index("c")
    base = wid * (NCHUNK * CH)
    pltpu.sync_copy(idx_hbm.at[wid], idx_v)
    gsem = (g0, g1)
    wsem = (w0, w1)

    def gathers(i, buf, sem):
        for g in range(G):
            pltpu.async_copy(
                table_hbm.at[idx_v.at[i * G + g]],
                big.at[buf].at[pl.ds(g * CH, CH)],
                sem,
            )

    def drain_gathers(i, buf, sem):
        for g in range(G):
            pltpu.make_async_copy(
                table_hbm.at[idx_v.at[i * G + g]],
                big.at[buf].at[pl.ds(g * CH, CH)],
                sem,
            ).wait()

    def write(i, buf, sem):
        return pltpu.make_async_copy(
            big.at[buf], out_hbm.at[pl.ds(base + i * ROWS_G, ROWS_G)], sem)

    gathers(0, 0, gsem[0])

    def body(i2, _):
        for buf in range(2):
            i = 2 * i2 + buf
            nbuf = 1 - buf

            @pl.when(i + 1 < NGRP)
            def _():
                @pl.when(i >= 1)
                def _():
                    write(i - 1, nbuf, wsem[nbuf]).wait()
                gathers(i + 1, nbuf, gsem[nbuf])

            drain_gathers(i, buf, gsem[buf])
            write(i, buf, wsem[buf]).start()
        return ()

    lax.fori_loop(0, NGRP // 2, body, (), unroll=False)
    write(NGRP - 2, 0, wsem[0]).wait()
    write(NGRP - 1, 1, wsem[1]).wait()


def kernel(input, weight):
    idx = input.astype(jnp.int32).reshape(NW, NCHUNK, CH)
    out = _emb_lookup(idx, weight)
    return out.reshape(BATCH, HIST, N_D)
